# baseline (speedup 1.0000x reference)
import functools

import jax
import jax.numpy as jnp
from jax import lax
from jax.experimental import pallas as pl
from jax.experimental.pallas import tpu as pltpu
from jax.experimental.pallas import tpu_sc as plsc

D = 64
B = 16384
NC, NS = 2, 16
NW = NC * NS
BPW = B // NW
L = 16


def _body(head_hbm, rel_hbm, tail_hbm, ent_hbm, relemb_hbm, out_hbm,
          hidx_v, ridx_v, tidx_v, h_v, r_v, t_v, q_v, out_v, sem):
    wid = lax.axis_index("s") * NC + lax.axis_index("c")
    base = wid * BPW

    pltpu.sync_copy(head_hbm.at[pl.ds(base, BPW)], hidx_v)
    pltpu.sync_copy(rel_hbm.at[pl.ds(base, BPW)], ridx_v)
    pltpu.sync_copy(tail_hbm.at[pl.ds(base, BPW)], tidx_v)

    ch = pltpu.async_copy(ent_hbm.at[hidx_v], h_v, sem)
    cr = pltpu.async_copy(relemb_hbm.at[ridx_v], r_v, sem)
    ct = pltpu.async_copy(ent_hbm.at[tidx_v], t_v, sem)
    ch.wait()
    cr.wait()
    ct.wait()

    @plsc.parallel_loop(0, BPW, 1, unroll=4)
    def _rowA(i):
        q = h_v[i, pl.ds(0, L)] * r_v[i, pl.ds(0, L)] * t_v[i, pl.ds(0, L)]
        for c in range(L, D, L):
            q += h_v[i, pl.ds(c, L)] * r_v[i, pl.ds(c, L)] * t_v[i, pl.ds(c, L)]
        q_v[pl.ds(i * L, L)] = q

    @plsc.parallel_loop(0, BPW // L, 1, unroll=2)
    def _rowB(g):
        rows = g * L + jnp.arange(L, dtype=jnp.int32)
        acc = plsc.load_gather(q_v, [rows * L])
        for l in range(1, L):
            acc += plsc.load_gather(q_v, [rows * L + l])
        out_v[pl.ds(g * L, L)] = acc

    pltpu.sync_copy(out_v, out_hbm.at[pl.ds(base, BPW)])


@jax.jit
def _distmult(head, relation, tail, entity_emb, relation_emb):
    mesh = plsc.VectorSubcoreMesh(core_axis_name="c", subcore_axis_name="s")
    return pl.kernel(
        _body,
        out_type=jax.ShapeDtypeStruct((B,), jnp.float32),
        mesh=mesh,
        scratch_types=[
            pltpu.VMEM((BPW,), jnp.int32),
            pltpu.VMEM((BPW,), jnp.int32),
            pltpu.VMEM((BPW,), jnp.int32),
            pltpu.VMEM((BPW, D), jnp.float32),
            pltpu.VMEM((BPW, D), jnp.float32),
            pltpu.VMEM((BPW, D), jnp.float32),
            pltpu.VMEM((BPW * L,), jnp.float32),
            pltpu.VMEM((BPW,), jnp.float32),
            pltpu.SemaphoreType.DMA,
        ],
        compiler_params=pltpu.CompilerParams(
            needs_layout_passes=False, use_tc_tiling_on_sc=False),
    )(head, relation, tail, entity_emb, relation_emb)


def kernel(head, relation, tail, entity_emb, relation_emb):
    return _distmult(head.astype(jnp.int32), relation.astype(jnp.int32),
                     tail.astype(jnp.int32), entity_emb, relation_emb)
